# Initial kernel scaffold; baseline (speedup 1.0000x reference)
import jax
import jax.numpy as jnp
from jax.experimental import pallas as pl


def kernel(x, target_feats, adj, W1, b1, W2, b2, L1W, L1b, L2W, L2b, L3W, L3b):
    raise NotImplementedError("write your pallas kernel here")


# speedup vs baseline: 6.2229x; 6.2229x over previous
import functools

import jax
import jax.numpy as jnp
from jax import lax
from jax.experimental import pallas as pl
from jax.experimental.pallas import tpu as pltpu
from jax.experimental.pallas import tpu_sc as plsc

N = 10000
E = 320000
NFEAT = 128
NHID = 128
NCAT = NFEAT + NHID

CHUNK = 128
NCHUNK = E // CHUNK
NC = 2
NS = 16
NW = NC * NS
BASE_CHUNKS = NCHUNK // NW
EXTRA_CHUNKS = NCHUNK - BASE_CHUNKS * NW
SLAB = (N // NS) // 8 * 8
TAIL = N - NS * SLAB

ROW_BLK = 1000
GRID = N // ROW_BLK


def _sc_agg_body(support_hbm, src_hbm, dst_hbm, zeros_hbm, out_hbm,
                 sidx_v, didx_v, rows_v, acc_sh, sem):
    cid = lax.axis_index("c")
    sid = lax.axis_index("s")
    wid = sid * NC + cid

    pltpu.sync_copy(zeros_hbm.at[pl.ds(sid * SLAB, SLAB)],
                    acc_sh.at[pl.ds(sid * SLAB, SLAB)])

    @pl.when(sid == 0)
    def _zero_tail():
        pltpu.sync_copy(zeros_hbm.at[pl.ds(NS * SLAB, TAIL)],
                        acc_sh.at[pl.ds(NS * SLAB, TAIL)])

    plsc.subcore_barrier()

    nk = BASE_CHUNKS + jnp.where(wid < EXTRA_CHUNKS, 1, 0)

    def body(k, carry):
        chunk = wid + k * NW
        pltpu.sync_copy(src_hbm.at[pl.ds(chunk * CHUNK, CHUNK)], sidx_v)
        pltpu.sync_copy(dst_hbm.at[pl.ds(chunk * CHUNK, CHUNK)], didx_v)
        pltpu.async_copy(support_hbm.at[sidx_v], rows_v, sem).wait()
        pltpu.sync_copy(rows_v, acc_sh.at[didx_v], add=True)
        return carry

    lax.fori_loop(0, nk, body, 0)
    plsc.subcore_barrier()

    pltpu.sync_copy(acc_sh.at[pl.ds(sid * SLAB, SLAB)],
                    out_hbm.at[cid, pl.ds(sid * SLAB, SLAB)])

    @pl.when(sid == 0)
    def _write_tail():
        pltpu.sync_copy(acc_sh.at[pl.ds(NS * SLAB, TAIL)],
                        out_hbm.at[cid, pl.ds(NS * SLAB, TAIL)])


_sc_agg = pl.kernel(
    _sc_agg_body,
    out_type=jax.ShapeDtypeStruct((NC, N, NFEAT), jnp.float32),
    mesh=plsc.VectorSubcoreMesh(core_axis_name="c", subcore_axis_name="s"),
    scratch_types=[
        pltpu.VMEM((CHUNK,), jnp.int32),
        pltpu.VMEM((CHUNK,), jnp.int32),
        pltpu.VMEM((CHUNK, NFEAT), jnp.float32),
        pltpu.VMEM_SHARED((N, NFEAT), jnp.float32),
        pltpu.SemaphoreType.DMA,
    ],
)


def _pre_body(x_ref, w_ref, o_ref):
    o_ref[...] = jnp.dot(x_ref[...], w_ref[...],
                         preferred_element_type=jnp.float32)


_pre = pl.pallas_call(
    _pre_body,
    grid=(GRID,),
    in_specs=[
        pl.BlockSpec((ROW_BLK, NFEAT), lambda i: (i, 0)),
        pl.BlockSpec((NFEAT, NHID), lambda i: (0, 0)),
    ],
    out_specs=pl.BlockSpec((ROW_BLK, NHID), lambda i: (i, 0)),
    out_shape=jax.ShapeDtypeStruct((N, NHID), jnp.float32),
)


def _mid_body(acc_ref, b1_ref, w2_ref, o_ref):
    h = jax.nn.relu(acc_ref[0] + acc_ref[1] + b1_ref[...])
    o_ref[...] = jnp.dot(h, w2_ref[...], preferred_element_type=jnp.float32)


_mid = pl.pallas_call(
    _mid_body,
    grid=(GRID,),
    in_specs=[
        pl.BlockSpec((NC, ROW_BLK, NHID), lambda i: (0, i, 0)),
        pl.BlockSpec((1, NHID), lambda i: (0, 0)),
        pl.BlockSpec((NHID, NHID), lambda i: (0, 0)),
    ],
    out_specs=pl.BlockSpec((ROW_BLK, NHID), lambda i: (i, 0)),
    out_shape=jax.ShapeDtypeStruct((N, NHID), jnp.float32),
)


def _post_body(acc_ref, b2_ref, tf_ref, l1w_ref, l1b_ref, l2w_ref, l2b_ref,
               l3w_ref, l3b_ref, o_ref):
    out3 = acc_ref[0] + acc_ref[1] + b2_ref[...]
    cat = jnp.concatenate([out3, tf_ref[...]], axis=1)
    h = jax.nn.relu(jnp.dot(cat, l1w_ref[...],
                            preferred_element_type=jnp.float32) + l1b_ref[...])
    h = jax.nn.relu(jnp.dot(h, l2w_ref[...],
                            preferred_element_type=jnp.float32) + l2b_ref[...])
    o = jnp.dot(h, l3w_ref[...],
                preferred_element_type=jnp.float32) + l3b_ref[...]
    m = jnp.max(o, axis=1, keepdims=True)
    s = jnp.sum(jnp.exp(o - m), axis=1, keepdims=True)
    o_ref[...] = o - m - jnp.log(s)


_post = pl.pallas_call(
    _post_body,
    grid=(GRID,),
    in_specs=[
        pl.BlockSpec((NC, ROW_BLK, NHID), lambda i: (0, i, 0)),
        pl.BlockSpec((1, NHID), lambda i: (0, 0)),
        pl.BlockSpec((ROW_BLK, NFEAT), lambda i: (i, 0)),
        pl.BlockSpec((NCAT, NCAT), lambda i: (0, 0)),
        pl.BlockSpec((1, NCAT), lambda i: (0, 0)),
        pl.BlockSpec((NCAT, NCAT), lambda i: (0, 0)),
        pl.BlockSpec((1, NCAT), lambda i: (0, 0)),
        pl.BlockSpec((NCAT, NFEAT), lambda i: (0, 0)),
        pl.BlockSpec((1, NFEAT), lambda i: (0, 0)),
    ],
    out_specs=pl.BlockSpec((ROW_BLK, NFEAT), lambda i: (i, 0)),
    out_shape=jax.ShapeDtypeStruct((N, NFEAT), jnp.float32),
)


def kernel(x, target_feats, adj, W1, b1, W2, b2, L1W, L1b, L2W, L2b, L3W, L3b):
    src1 = adj[0, 0]
    dst1 = adj[0, 1]
    src2 = adj[1, 0]
    dst2 = adj[1, 1]
    zeros = jnp.zeros((N, NFEAT), jnp.float32)

    support1 = _pre(x, W1)
    acc1 = _sc_agg(support1, src1, dst1, zeros)
    support2 = _mid(acc1, b1.reshape(1, NHID), W2)
    acc2 = _sc_agg(support2, src2, dst2, zeros)
    return _post(acc2, b2.reshape(1, NHID), target_feats,
                 L1W, L1b.reshape(1, NCAT), L2W, L2b.reshape(1, NCAT),
                 L3W, L3b.reshape(1, NFEAT))


# speedup vs baseline: 10.4579x; 1.6805x over previous
import functools

import jax
import jax.numpy as jnp
from jax import lax
from jax.experimental import pallas as pl
from jax.experimental.pallas import tpu as pltpu
from jax.experimental.pallas import tpu_sc as plsc

N = 10000
E = 320000
NFEAT = 128
NHID = 128
NCAT = NFEAT + NHID

CHUNK = 128
NCHUNK = E // CHUNK
NC = 2
NS = 16
NW = NC * NS
BASE_CHUNKS = NCHUNK // NW
EXTRA_CHUNKS = NCHUNK - BASE_CHUNKS * NW
SLAB = (N // NS) // 8 * 8
TAIL = N - NS * SLAB

ROW_BLK = 1000
GRID = N // ROW_BLK


def _sc_agg_body(support_hbm, src_hbm, dst_hbm, zeros_hbm, out_hbm,
                 sidx_all, sidx_x, didx0, didx1, rows0, rows1,
                 acc_sh, sem_i0, sem_i1, sem_g0, sem_g1, sem_s0, sem_s1):
    cid = lax.axis_index("c")
    sid = lax.axis_index("s")
    wid = sid * NC + cid
    didx = (didx0, didx1)
    rows = (rows0, rows1)
    sem_i = (sem_i0, sem_i1)
    sem_g = (sem_g0, sem_g1)
    sem_s = (sem_s0, sem_s1)

    cstart = wid * BASE_CHUNKS + jnp.minimum(wid, EXTRA_CHUNKS)

    pltpu.sync_copy(zeros_hbm.at[pl.ds(sid * SLAB, SLAB)],
                    acc_sh.at[pl.ds(sid * SLAB, SLAB)])

    @pl.when(sid == 0)
    def _zero_tail():
        pltpu.sync_copy(zeros_hbm.at[pl.ds(NS * SLAB, TAIL)],
                        acc_sh.at[pl.ds(NS * SLAB, TAIL)])

    pltpu.sync_copy(src_hbm.at[pl.ds(cstart * CHUNK, BASE_CHUNKS * CHUNK)],
                    sidx_all)
    plsc.subcore_barrier()

    @pl.loop(0, BASE_CHUNKS, step=2)
    def _ring(g):
        for b in range(2):
            k = g + b

            @pl.when(k >= 2)
            def _drain():
                pltpu.make_async_copy(
                    rows[b], acc_sh.at[didx[b]], sem_s[b]).wait()

            pltpu.async_copy(
                dst_hbm.at[pl.ds((cstart + k) * CHUNK, CHUNK)],
                didx[b], sem_i[b])
            gather = pltpu.async_copy(
                support_hbm.at[sidx_all.at[pl.ds(k * CHUNK, CHUNK)]],
                rows[b], sem_g[b])
            gather.wait()
            pltpu.make_async_copy(
                dst_hbm.at[pl.ds((cstart + k) * CHUNK, CHUNK)],
                didx[b], sem_i[b]).wait()
            pltpu.async_copy(rows[b], acc_sh.at[didx[b]], sem_s[b], add=True)

    for b in range(2):
        pltpu.make_async_copy(rows[b], acc_sh.at[didx[b]], sem_s[b]).wait()

    @pl.when(wid < EXTRA_CHUNKS)
    def _extra():
        chunk = cstart + BASE_CHUNKS
        pltpu.sync_copy(src_hbm.at[pl.ds(chunk * CHUNK, CHUNK)], sidx_x)
        pltpu.sync_copy(dst_hbm.at[pl.ds(chunk * CHUNK, CHUNK)], didx0)
        pltpu.async_copy(support_hbm.at[sidx_x], rows0, sem_g0).wait()
        pltpu.sync_copy(rows0, acc_sh.at[didx0], add=True)

    plsc.subcore_barrier()

    pltpu.sync_copy(acc_sh.at[pl.ds(sid * SLAB, SLAB)],
                    out_hbm.at[cid, pl.ds(sid * SLAB, SLAB)])

    @pl.when(sid == 0)
    def _write_tail():
        pltpu.sync_copy(acc_sh.at[pl.ds(NS * SLAB, TAIL)],
                        out_hbm.at[cid, pl.ds(NS * SLAB, TAIL)])


_sc_agg = pl.kernel(
    _sc_agg_body,
    out_type=jax.ShapeDtypeStruct((NC, N, NFEAT), jnp.float32),
    mesh=plsc.VectorSubcoreMesh(core_axis_name="c", subcore_axis_name="s"),
    scratch_types=[
        pltpu.VMEM((BASE_CHUNKS * CHUNK,), jnp.int32),
        pltpu.VMEM((CHUNK,), jnp.int32),
        pltpu.VMEM((CHUNK,), jnp.int32),
        pltpu.VMEM((CHUNK,), jnp.int32),
        pltpu.VMEM((CHUNK, NFEAT), jnp.float32),
        pltpu.VMEM((CHUNK, NFEAT), jnp.float32),
        pltpu.VMEM_SHARED((N, NFEAT), jnp.float32),
        pltpu.SemaphoreType.DMA,
        pltpu.SemaphoreType.DMA,
        pltpu.SemaphoreType.DMA,
        pltpu.SemaphoreType.DMA,
        pltpu.SemaphoreType.DMA,
        pltpu.SemaphoreType.DMA,
    ],
)


def _pre_body(x_ref, w_ref, o_ref):
    o_ref[...] = jnp.dot(x_ref[...], w_ref[...],
                         preferred_element_type=jnp.float32)


_pre = pl.pallas_call(
    _pre_body,
    grid=(GRID,),
    in_specs=[
        pl.BlockSpec((ROW_BLK, NFEAT), lambda i: (i, 0)),
        pl.BlockSpec((NFEAT, NHID), lambda i: (0, 0)),
    ],
    out_specs=pl.BlockSpec((ROW_BLK, NHID), lambda i: (i, 0)),
    out_shape=jax.ShapeDtypeStruct((N, NHID), jnp.float32),
)


def _mid_body(acc_ref, b1_ref, w2_ref, o_ref):
    h = jax.nn.relu(acc_ref[0] + acc_ref[1] + b1_ref[...])
    o_ref[...] = jnp.dot(h, w2_ref[...], preferred_element_type=jnp.float32)


_mid = pl.pallas_call(
    _mid_body,
    grid=(GRID,),
    in_specs=[
        pl.BlockSpec((NC, ROW_BLK, NHID), lambda i: (0, i, 0)),
        pl.BlockSpec((1, NHID), lambda i: (0, 0)),
        pl.BlockSpec((NHID, NHID), lambda i: (0, 0)),
    ],
    out_specs=pl.BlockSpec((ROW_BLK, NHID), lambda i: (i, 0)),
    out_shape=jax.ShapeDtypeStruct((N, NHID), jnp.float32),
)


def _post_body(acc_ref, b2_ref, tf_ref, l1w_ref, l1b_ref, l2w_ref, l2b_ref,
               l3w_ref, l3b_ref, o_ref):
    out3 = acc_ref[0] + acc_ref[1] + b2_ref[...]
    cat = jnp.concatenate([out3, tf_ref[...]], axis=1)
    h = jax.nn.relu(jnp.dot(cat, l1w_ref[...],
                            preferred_element_type=jnp.float32) + l1b_ref[...])
    h = jax.nn.relu(jnp.dot(h, l2w_ref[...],
                            preferred_element_type=jnp.float32) + l2b_ref[...])
    o = jnp.dot(h, l3w_ref[...],
                preferred_element_type=jnp.float32) + l3b_ref[...]
    m = jnp.max(o, axis=1, keepdims=True)
    s = jnp.sum(jnp.exp(o - m), axis=1, keepdims=True)
    o_ref[...] = o - m - jnp.log(s)


_post = pl.pallas_call(
    _post_body,
    grid=(GRID,),
    in_specs=[
        pl.BlockSpec((NC, ROW_BLK, NHID), lambda i: (0, i, 0)),
        pl.BlockSpec((1, NHID), lambda i: (0, 0)),
        pl.BlockSpec((ROW_BLK, NFEAT), lambda i: (i, 0)),
        pl.BlockSpec((NCAT, NCAT), lambda i: (0, 0)),
        pl.BlockSpec((1, NCAT), lambda i: (0, 0)),
        pl.BlockSpec((NCAT, NCAT), lambda i: (0, 0)),
        pl.BlockSpec((1, NCAT), lambda i: (0, 0)),
        pl.BlockSpec((NCAT, NFEAT), lambda i: (0, 0)),
        pl.BlockSpec((1, NFEAT), lambda i: (0, 0)),
    ],
    out_specs=pl.BlockSpec((ROW_BLK, NFEAT), lambda i: (i, 0)),
    out_shape=jax.ShapeDtypeStruct((N, NFEAT), jnp.float32),
)


def kernel(x, target_feats, adj, W1, b1, W2, b2, L1W, L1b, L2W, L2b, L3W, L3b):
    src1 = adj[0, 0]
    dst1 = adj[0, 1]
    src2 = adj[1, 0]
    dst2 = adj[1, 1]
    zeros = jnp.zeros((N, NFEAT), jnp.float32)

    support1 = _pre(x, W1)
    acc1 = _sc_agg(support1, src1, dst1, zeros)
    support2 = _mid(acc1, b1.reshape(1, NHID), W2)
    acc2 = _sc_agg(support2, src2, dst2, zeros)
    return _post(acc2, b2.reshape(1, NHID), target_feats,
                 L1W, L1b.reshape(1, NCAT), L2W, L2b.reshape(1, NCAT),
                 L3W, L3b.reshape(1, NFEAT))


# speedup vs baseline: 12.1339x; 1.1603x over previous
import functools

import jax
import jax.numpy as jnp
from jax import lax
from jax.experimental import pallas as pl
from jax.experimental.pallas import tpu as pltpu
from jax.experimental.pallas import tpu_sc as plsc

N = 10000
E = 320000
NFEAT = 128
NHID = 128
NCAT = NFEAT + NHID

CHUNK = 128
NCHUNK = E // CHUNK
NC = 2
NS = 16
NW = NC * NS
BASE_CHUNKS = NCHUNK // NW
EXTRA_CHUNKS = NCHUNK - BASE_CHUNKS * NW
SLAB = (N // NS) // 8 * 8
TAIL = N - NS * SLAB

ROW_BLK = 1000
GRID = N // ROW_BLK


NBUF = 3


def _sc_agg_body(support_hbm, src_hbm, dst_hbm, zeros_hbm, out_hbm,
                 sidx0, sidx1, sidx2, didx0, didx1, didx2,
                 rows0, rows1, rows2, acc_sh,
                 sem_a0, sem_a1, sem_a2, sem_b0, sem_b1, sem_b2,
                 sem_g0, sem_g1, sem_g2, sem_s0, sem_s1, sem_s2):
    cid = lax.axis_index("c")
    sid = lax.axis_index("s")
    wid = sid * NC + cid
    sidx = (sidx0, sidx1, sidx2)
    didx = (didx0, didx1, didx2)
    rows = (rows0, rows1, rows2)
    sem_a = (sem_a0, sem_a1, sem_a2)
    sem_b = (sem_b0, sem_b1, sem_b2)
    sem_g = (sem_g0, sem_g1, sem_g2)
    sem_s = (sem_s0, sem_s1, sem_s2)

    cstart = wid * BASE_CHUNKS + jnp.minimum(wid, EXTRA_CHUNKS)

    pltpu.sync_copy(zeros_hbm.at[pl.ds(sid * SLAB, SLAB)],
                    acc_sh.at[pl.ds(sid * SLAB, SLAB)])

    @pl.when(sid == 0)
    def _zero_tail():
        pltpu.sync_copy(zeros_hbm.at[pl.ds(NS * SLAB, TAIL)],
                        acc_sh.at[pl.ds(NS * SLAB, TAIL)])

    plsc.subcore_barrier()

    def _issue_idx(kk, b):
        pltpu.async_copy(src_hbm.at[pl.ds((cstart + kk) * CHUNK, CHUNK)],
                         sidx[b], sem_a[b])
        pltpu.async_copy(dst_hbm.at[pl.ds((cstart + kk) * CHUNK, CHUNK)],
                         didx[b], sem_b[b])

    def _issue_gather(b):
        pltpu.make_async_copy(src_hbm.at[pl.ds(0, CHUNK)], sidx[b],
                              sem_a[b]).wait()
        pltpu.async_copy(support_hbm.at[sidx[b]], rows[b], sem_g[b])

    _issue_idx(0, 0)
    _issue_idx(1, 1)
    _issue_gather(0)

    @pl.loop(0, BASE_CHUNKS, step=NBUF)
    def _ring(g):
        for b in range(NBUF):
            k = g + b
            b1 = (b + 1) % NBUF
            b2 = (b + 2) % NBUF

            @pl.when(k >= 1)
            def _drain():
                pltpu.make_async_copy(
                    rows[b2], acc_sh.at[didx[b2]], sem_s[b2]).wait()

            @pl.when(k + 2 < BASE_CHUNKS)
            def _ahead_idx():
                _issue_idx(k + 2, b2)

            @pl.when(k + 1 < BASE_CHUNKS)
            def _ahead_gather():
                _issue_gather(b1)

            pltpu.make_async_copy(support_hbm.at[sidx[b]], rows[b],
                                  sem_g[b]).wait()
            pltpu.make_async_copy(dst_hbm.at[pl.ds(0, CHUNK)], didx[b],
                                  sem_b[b]).wait()
            pltpu.async_copy(rows[b], acc_sh.at[didx[b]], sem_s[b], add=True)

    bl = (BASE_CHUNKS - 1) % NBUF
    pltpu.make_async_copy(rows[bl], acc_sh.at[didx[bl]], sem_s[bl]).wait()

    @pl.when(wid < EXTRA_CHUNKS)
    def _extra():
        chunk = cstart + BASE_CHUNKS
        pltpu.sync_copy(src_hbm.at[pl.ds(chunk * CHUNK, CHUNK)], sidx0)
        pltpu.sync_copy(dst_hbm.at[pl.ds(chunk * CHUNK, CHUNK)], didx0)
        pltpu.async_copy(support_hbm.at[sidx0], rows0, sem_g0).wait()
        pltpu.sync_copy(rows0, acc_sh.at[didx0], add=True)

    plsc.subcore_barrier()

    pltpu.sync_copy(acc_sh.at[pl.ds(sid * SLAB, SLAB)],
                    out_hbm.at[cid, pl.ds(sid * SLAB, SLAB)])

    @pl.when(sid == 0)
    def _write_tail():
        pltpu.sync_copy(acc_sh.at[pl.ds(NS * SLAB, TAIL)],
                        out_hbm.at[cid, pl.ds(NS * SLAB, TAIL)])


_sc_agg = pl.kernel(
    _sc_agg_body,
    out_type=jax.ShapeDtypeStruct((NC, N, NFEAT), jnp.float32),
    mesh=plsc.VectorSubcoreMesh(core_axis_name="c", subcore_axis_name="s"),
    scratch_types=(
        [pltpu.VMEM((CHUNK,), jnp.int32)] * 6
        + [pltpu.VMEM((CHUNK, NFEAT), jnp.float32)] * 3
        + [pltpu.VMEM_SHARED((N, NFEAT), jnp.float32)]
        + [pltpu.SemaphoreType.DMA] * 12
    ),
)


def _pre_body(x_ref, w_ref, o_ref):
    o_ref[...] = jnp.dot(x_ref[...], w_ref[...],
                         preferred_element_type=jnp.float32)


_pre = pl.pallas_call(
    _pre_body,
    grid=(GRID,),
    in_specs=[
        pl.BlockSpec((ROW_BLK, NFEAT), lambda i: (i, 0)),
        pl.BlockSpec((NFEAT, NHID), lambda i: (0, 0)),
    ],
    out_specs=pl.BlockSpec((ROW_BLK, NHID), lambda i: (i, 0)),
    out_shape=jax.ShapeDtypeStruct((N, NHID), jnp.float32),
)


def _mid_body(acc_ref, b1_ref, w2_ref, o_ref):
    h = jax.nn.relu(acc_ref[0] + acc_ref[1] + b1_ref[...])
    o_ref[...] = jnp.dot(h, w2_ref[...], preferred_element_type=jnp.float32)


_mid = pl.pallas_call(
    _mid_body,
    grid=(GRID,),
    in_specs=[
        pl.BlockSpec((NC, ROW_BLK, NHID), lambda i: (0, i, 0)),
        pl.BlockSpec((1, NHID), lambda i: (0, 0)),
        pl.BlockSpec((NHID, NHID), lambda i: (0, 0)),
    ],
    out_specs=pl.BlockSpec((ROW_BLK, NHID), lambda i: (i, 0)),
    out_shape=jax.ShapeDtypeStruct((N, NHID), jnp.float32),
)


def _post_body(acc_ref, b2_ref, tf_ref, l1w_ref, l1b_ref, l2w_ref, l2b_ref,
               l3w_ref, l3b_ref, o_ref):
    out3 = acc_ref[0] + acc_ref[1] + b2_ref[...]
    cat = jnp.concatenate([out3, tf_ref[...]], axis=1)
    h = jax.nn.relu(jnp.dot(cat, l1w_ref[...],
                            preferred_element_type=jnp.float32) + l1b_ref[...])
    h = jax.nn.relu(jnp.dot(h, l2w_ref[...],
                            preferred_element_type=jnp.float32) + l2b_ref[...])
    o = jnp.dot(h, l3w_ref[...],
                preferred_element_type=jnp.float32) + l3b_ref[...]
    m = jnp.max(o, axis=1, keepdims=True)
    s = jnp.sum(jnp.exp(o - m), axis=1, keepdims=True)
    o_ref[...] = o - m - jnp.log(s)


_post = pl.pallas_call(
    _post_body,
    grid=(GRID,),
    in_specs=[
        pl.BlockSpec((NC, ROW_BLK, NHID), lambda i: (0, i, 0)),
        pl.BlockSpec((1, NHID), lambda i: (0, 0)),
        pl.BlockSpec((ROW_BLK, NFEAT), lambda i: (i, 0)),
        pl.BlockSpec((NCAT, NCAT), lambda i: (0, 0)),
        pl.BlockSpec((1, NCAT), lambda i: (0, 0)),
        pl.BlockSpec((NCAT, NCAT), lambda i: (0, 0)),
        pl.BlockSpec((1, NCAT), lambda i: (0, 0)),
        pl.BlockSpec((NCAT, NFEAT), lambda i: (0, 0)),
        pl.BlockSpec((1, NFEAT), lambda i: (0, 0)),
    ],
    out_specs=pl.BlockSpec((ROW_BLK, NFEAT), lambda i: (i, 0)),
    out_shape=jax.ShapeDtypeStruct((N, NFEAT), jnp.float32),
)


def kernel(x, target_feats, adj, W1, b1, W2, b2, L1W, L1b, L2W, L2b, L3W, L3b):
    src1 = adj[0, 0]
    dst1 = adj[0, 1]
    src2 = adj[1, 0]
    dst2 = adj[1, 1]
    zeros = jnp.zeros((N, NFEAT), jnp.float32)

    support1 = _pre(x, W1)
    acc1 = _sc_agg(support1, src1, dst1, zeros)
    support2 = _mid(acc1, b1.reshape(1, NHID), W2)
    acc2 = _sc_agg(support2, src2, dst2, zeros)
    return _post(acc2, b2.reshape(1, NHID), target_feats,
                 L1W, L1b.reshape(1, NCAT), L2W, L2b.reshape(1, NCAT),
                 L3W, L3b.reshape(1, NFEAT))


# speedup vs baseline: 13.3504x; 1.1003x over previous
import functools

import jax
import jax.numpy as jnp
from jax import lax
from jax.experimental import pallas as pl
from jax.experimental.pallas import tpu as pltpu
from jax.experimental.pallas import tpu_sc as plsc

N = 10000
E = 320000
NFEAT = 128
NHID = 128
NCAT = NFEAT + NHID

CHUNK = 128
NCHUNK = E // CHUNK
NC = 2
NS = 16
NW = NC * NS
BASE_CHUNKS = NCHUNK // NW
EXTRA_CHUNKS = NCHUNK - BASE_CHUNKS * NW
SLAB = (N // NS) // 8 * 8
TAIL = N - NS * SLAB

ROW_BLK = 1000
GRID = N // ROW_BLK


NBUF = 3


def _sc_agg_body(lay, support_hbm, adj_hbm, zeros_hbm, out_hbm,
                 sidx0, sidx1, sidx2, didx0, didx1, didx2,
                 rows0, rows1, rows2, acc_sh,
                 sem_a0, sem_a1, sem_a2, sem_b0, sem_b1, sem_b2,
                 sem_g0, sem_g1, sem_g2, sem_s0, sem_s1, sem_s2):
    src_hbm = adj_hbm.at[lay, 0]
    dst_hbm = adj_hbm.at[lay, 1]
    cid = lax.axis_index("c")
    sid = lax.axis_index("s")
    wid = sid * NC + cid
    sidx = (sidx0, sidx1, sidx2)
    didx = (didx0, didx1, didx2)
    rows = (rows0, rows1, rows2)
    sem_a = (sem_a0, sem_a1, sem_a2)
    sem_b = (sem_b0, sem_b1, sem_b2)
    sem_g = (sem_g0, sem_g1, sem_g2)
    sem_s = (sem_s0, sem_s1, sem_s2)

    cstart = wid * BASE_CHUNKS + jnp.minimum(wid, EXTRA_CHUNKS)

    pltpu.sync_copy(zeros_hbm.at[pl.ds(sid * SLAB, SLAB)],
                    acc_sh.at[pl.ds(sid * SLAB, SLAB)])

    @pl.when(sid == 0)
    def _zero_tail():
        pltpu.sync_copy(zeros_hbm.at[pl.ds(NS * SLAB, TAIL)],
                        acc_sh.at[pl.ds(NS * SLAB, TAIL)])

    plsc.subcore_barrier()

    def _issue_idx(kk, b):
        pltpu.async_copy(src_hbm.at[pl.ds((cstart + kk) * CHUNK, CHUNK)],
                         sidx[b], sem_a[b])
        pltpu.async_copy(dst_hbm.at[pl.ds((cstart + kk) * CHUNK, CHUNK)],
                         didx[b], sem_b[b])

    def _issue_gather(b):
        pltpu.make_async_copy(src_hbm.at[pl.ds(0, CHUNK)], sidx[b],
                              sem_a[b]).wait()
        pltpu.async_copy(support_hbm.at[sidx[b]], rows[b], sem_g[b])

    _issue_idx(0, 0)
    _issue_idx(1, 1)
    _issue_gather(0)

    @pl.loop(0, BASE_CHUNKS, step=NBUF)
    def _ring(g):
        for b in range(NBUF):
            k = g + b
            b1 = (b + 1) % NBUF
            b2 = (b + 2) % NBUF

            @pl.when(k + 1 < BASE_CHUNKS)
            def _ahead_gather():
                _issue_gather(b1)

            @pl.when(k >= 1)
            def _drain():
                pltpu.make_async_copy(
                    rows[b2], acc_sh.at[didx[b2]], sem_s[b2]).wait()

            @pl.when(k + 2 < BASE_CHUNKS)
            def _ahead_idx():
                _issue_idx(k + 2, b2)

            pltpu.make_async_copy(support_hbm.at[sidx[b]], rows[b],
                                  sem_g[b]).wait()
            pltpu.make_async_copy(dst_hbm.at[pl.ds(0, CHUNK)], didx[b],
                                  sem_b[b]).wait()
            pltpu.async_copy(rows[b], acc_sh.at[didx[b]], sem_s[b], add=True)

    bl = (BASE_CHUNKS - 1) % NBUF
    pltpu.make_async_copy(rows[bl], acc_sh.at[didx[bl]], sem_s[bl]).wait()

    @pl.when(wid < EXTRA_CHUNKS)
    def _extra():
        chunk = cstart + BASE_CHUNKS
        pltpu.sync_copy(src_hbm.at[pl.ds(chunk * CHUNK, CHUNK)], sidx0)
        pltpu.sync_copy(dst_hbm.at[pl.ds(chunk * CHUNK, CHUNK)], didx0)
        pltpu.async_copy(support_hbm.at[sidx0], rows0, sem_g0).wait()
        pltpu.sync_copy(rows0, acc_sh.at[didx0], add=True)

    plsc.subcore_barrier()

    pltpu.sync_copy(acc_sh.at[pl.ds(sid * SLAB, SLAB)],
                    out_hbm.at[cid, pl.ds(sid * SLAB, SLAB)])

    @pl.when(sid == 0)
    def _write_tail():
        pltpu.sync_copy(acc_sh.at[pl.ds(NS * SLAB, TAIL)],
                        out_hbm.at[cid, pl.ds(NS * SLAB, TAIL)])


def _make_sc_agg(lay):
    return pl.kernel(
        functools.partial(_sc_agg_body, lay),
        out_type=jax.ShapeDtypeStruct((NC, N, NFEAT), jnp.float32),
        mesh=plsc.VectorSubcoreMesh(core_axis_name="c", subcore_axis_name="s"),
        scratch_types=(
            [pltpu.VMEM((CHUNK,), jnp.int32)] * 6
            + [pltpu.VMEM((CHUNK, NFEAT), jnp.float32)] * 3
            + [pltpu.VMEM_SHARED((N, NFEAT), jnp.float32)]
            + [pltpu.SemaphoreType.DMA] * 12
        ),
    )


_sc_agg0 = _make_sc_agg(0)
_sc_agg1 = _make_sc_agg(1)


def _pre_body(x_ref, w_ref, o_ref):
    o_ref[...] = jnp.dot(x_ref[...], w_ref[...],
                         preferred_element_type=jnp.float32)


_pre = pl.pallas_call(
    _pre_body,
    grid=(GRID,),
    in_specs=[
        pl.BlockSpec((ROW_BLK, NFEAT), lambda i: (i, 0)),
        pl.BlockSpec((NFEAT, NHID), lambda i: (0, 0)),
    ],
    out_specs=pl.BlockSpec((ROW_BLK, NHID), lambda i: (i, 0)),
    out_shape=jax.ShapeDtypeStruct((N, NHID), jnp.float32),
)


def _mid_body(acc_ref, b1_ref, w2_ref, o_ref):
    h = jax.nn.relu(acc_ref[0] + acc_ref[1] + b1_ref[...])
    o_ref[...] = jnp.dot(h, w2_ref[...], preferred_element_type=jnp.float32)


_mid = pl.pallas_call(
    _mid_body,
    grid=(GRID,),
    in_specs=[
        pl.BlockSpec((NC, ROW_BLK, NHID), lambda i: (0, i, 0)),
        pl.BlockSpec((1, NHID), lambda i: (0, 0)),
        pl.BlockSpec((NHID, NHID), lambda i: (0, 0)),
    ],
    out_specs=pl.BlockSpec((ROW_BLK, NHID), lambda i: (i, 0)),
    out_shape=jax.ShapeDtypeStruct((N, NHID), jnp.float32),
)


def _post_body(acc_ref, b2_ref, tf_ref, l1w_ref, l1b_ref, l2w_ref, l2b_ref,
               l3w_ref, l3b_ref, o_ref):
    out3 = acc_ref[0] + acc_ref[1] + b2_ref[...]
    cat = jnp.concatenate([out3, tf_ref[...]], axis=1)
    h = jax.nn.relu(jnp.dot(cat, l1w_ref[...],
                            preferred_element_type=jnp.float32) + l1b_ref[...])
    h = jax.nn.relu(jnp.dot(h, l2w_ref[...],
                            preferred_element_type=jnp.float32) + l2b_ref[...])
    o = jnp.dot(h, l3w_ref[...],
                preferred_element_type=jnp.float32) + l3b_ref[...]
    m = jnp.max(o, axis=1, keepdims=True)
    s = jnp.sum(jnp.exp(o - m), axis=1, keepdims=True)
    o_ref[...] = o - m - jnp.log(s)


_post = pl.pallas_call(
    _post_body,
    grid=(GRID,),
    in_specs=[
        pl.BlockSpec((NC, ROW_BLK, NHID), lambda i: (0, i, 0)),
        pl.BlockSpec((1, NHID), lambda i: (0, 0)),
        pl.BlockSpec((ROW_BLK, NFEAT), lambda i: (i, 0)),
        pl.BlockSpec((NCAT, NCAT), lambda i: (0, 0)),
        pl.BlockSpec((1, NCAT), lambda i: (0, 0)),
        pl.BlockSpec((NCAT, NCAT), lambda i: (0, 0)),
        pl.BlockSpec((1, NCAT), lambda i: (0, 0)),
        pl.BlockSpec((NCAT, NFEAT), lambda i: (0, 0)),
        pl.BlockSpec((1, NFEAT), lambda i: (0, 0)),
    ],
    out_specs=pl.BlockSpec((ROW_BLK, NFEAT), lambda i: (i, 0)),
    out_shape=jax.ShapeDtypeStruct((N, NFEAT), jnp.float32),
)


def kernel(x, target_feats, adj, W1, b1, W2, b2, L1W, L1b, L2W, L2b, L3W, L3b):
    zeros = jnp.zeros((N, NFEAT), jnp.float32)

    support1 = _pre(x, W1)
    acc1 = _sc_agg0(support1, adj, zeros)
    support2 = _mid(acc1, b1.reshape(1, NHID), W2)
    acc2 = _sc_agg1(support2, adj, zeros)
    return _post(acc2, b2.reshape(1, NHID), target_feats,
                 L1W, L1b.reshape(1, NCAT), L2W, L2b.reshape(1, NCAT),
                 L3W, L3b.reshape(1, NFEAT))


# speedup vs baseline: 13.3641x; 1.0010x over previous
import functools

import jax
import jax.numpy as jnp
from jax import lax
from jax.experimental import pallas as pl
from jax.experimental.pallas import tpu as pltpu
from jax.experimental.pallas import tpu_sc as plsc

N = 10000
E = 320000
NFEAT = 128
NHID = 128
NCAT = NFEAT + NHID

CHUNK = 128
NCHUNK = E // CHUNK
NC = 2
NS = 16
NW = NC * NS
BASE_CHUNKS = NCHUNK // NW
EXTRA_CHUNKS = NCHUNK - BASE_CHUNKS * NW
SLAB = (N // NS) // 8 * 8
TAIL = N - NS * SLAB

ROW_BLK = 1000
GRID = N // ROW_BLK


NBUF = 3


def _sc_agg_body(lay, support_hbm, adj_hbm, zeros_hbm, out_hbm,
                 sidx0, sidx1, sidx2, didx0, didx1, didx2,
                 rows0, rows1, rows2, acc_sh,
                 sem_a0, sem_a1, sem_a2, sem_b0, sem_b1, sem_b2,
                 sem_g0, sem_g1, sem_g2, sem_s0, sem_s1, sem_s2):
    src_hbm = adj_hbm.at[lay, 0]
    dst_hbm = adj_hbm.at[lay, 1]
    cid = lax.axis_index("c")
    sid = lax.axis_index("s")
    wid = sid * NC + cid
    sidx = (sidx0, sidx1, sidx2)
    didx = (didx0, didx1, didx2)
    rows = (rows0, rows1, rows2)
    sem_a = (sem_a0, sem_a1, sem_a2)
    sem_b = (sem_b0, sem_b1, sem_b2)
    sem_g = (sem_g0, sem_g1, sem_g2)
    sem_s = (sem_s0, sem_s1, sem_s2)

    cstart = wid * BASE_CHUNKS + jnp.minimum(wid, EXTRA_CHUNKS)

    def _issue_idx(kk, b):
        pltpu.async_copy(src_hbm.at[pl.ds((cstart + kk) * CHUNK, CHUNK)],
                         sidx[b], sem_a[b])
        pltpu.async_copy(dst_hbm.at[pl.ds((cstart + kk) * CHUNK, CHUNK)],
                         didx[b], sem_b[b])

    def _issue_gather(b):
        pltpu.make_async_copy(src_hbm.at[pl.ds(0, CHUNK)], sidx[b],
                              sem_a[b]).wait()
        pltpu.async_copy(support_hbm.at[sidx[b]], rows[b], sem_g[b])

    _issue_idx(0, 0)
    _issue_idx(1, 1)
    _issue_gather(0)

    pltpu.sync_copy(zeros_hbm, acc_sh.at[pl.ds(sid * SLAB, SLAB)])

    @pl.when(sid == 0)
    def _zero_tail():
        pltpu.sync_copy(zeros_hbm.at[pl.ds(0, TAIL)],
                        acc_sh.at[pl.ds(NS * SLAB, TAIL)])

    plsc.subcore_barrier()

    @pl.loop(0, BASE_CHUNKS, step=NBUF)
    def _ring(g):
        for b in range(NBUF):
            k = g + b
            b1 = (b + 1) % NBUF
            b2 = (b + 2) % NBUF

            @pl.when(k + 1 < BASE_CHUNKS)
            def _ahead_gather():
                _issue_gather(b1)

            @pl.when(k >= 1)
            def _drain():
                pltpu.make_async_copy(
                    rows[b2], acc_sh.at[didx[b2]], sem_s[b2]).wait()

            @pl.when(k + 2 < BASE_CHUNKS)
            def _ahead_idx():
                _issue_idx(k + 2, b2)

            pltpu.make_async_copy(support_hbm.at[sidx[b]], rows[b],
                                  sem_g[b]).wait()
            pltpu.make_async_copy(dst_hbm.at[pl.ds(0, CHUNK)], didx[b],
                                  sem_b[b]).wait()
            pltpu.async_copy(rows[b], acc_sh.at[didx[b]], sem_s[b], add=True)

    bl = (BASE_CHUNKS - 1) % NBUF
    pltpu.make_async_copy(rows[bl], acc_sh.at[didx[bl]], sem_s[bl]).wait()

    @pl.when(wid < EXTRA_CHUNKS)
    def _extra():
        chunk = cstart + BASE_CHUNKS
        pltpu.sync_copy(src_hbm.at[pl.ds(chunk * CHUNK, CHUNK)], sidx0)
        pltpu.sync_copy(dst_hbm.at[pl.ds(chunk * CHUNK, CHUNK)], didx0)
        pltpu.async_copy(support_hbm.at[sidx0], rows0, sem_g0).wait()
        pltpu.sync_copy(rows0, acc_sh.at[didx0], add=True)

    plsc.subcore_barrier()

    pltpu.sync_copy(acc_sh.at[pl.ds(sid * SLAB, SLAB)],
                    out_hbm.at[cid, pl.ds(sid * SLAB, SLAB)])

    @pl.when(sid == 0)
    def _write_tail():
        pltpu.sync_copy(acc_sh.at[pl.ds(NS * SLAB, TAIL)],
                        out_hbm.at[cid, pl.ds(NS * SLAB, TAIL)])


def _make_sc_agg(lay):
    return pl.kernel(
        functools.partial(_sc_agg_body, lay),
        out_type=jax.ShapeDtypeStruct((NC, N, NFEAT), jnp.float32),
        mesh=plsc.VectorSubcoreMesh(core_axis_name="c", subcore_axis_name="s"),
        scratch_types=(
            [pltpu.VMEM((CHUNK,), jnp.int32)] * 6
            + [pltpu.VMEM((CHUNK, NFEAT), jnp.float32)] * 3
            + [pltpu.VMEM_SHARED((N, NFEAT), jnp.float32)]
            + [pltpu.SemaphoreType.DMA] * 12
        ),
    )


_sc_agg0 = _make_sc_agg(0)
_sc_agg1 = _make_sc_agg(1)


def _pre_body(x_ref, w_ref, o_ref):
    o_ref[...] = jnp.dot(x_ref[...], w_ref[...],
                         preferred_element_type=jnp.float32)


_pre = pl.pallas_call(
    _pre_body,
    grid=(GRID,),
    in_specs=[
        pl.BlockSpec((ROW_BLK, NFEAT), lambda i: (i, 0)),
        pl.BlockSpec((NFEAT, NHID), lambda i: (0, 0)),
    ],
    out_specs=pl.BlockSpec((ROW_BLK, NHID), lambda i: (i, 0)),
    out_shape=jax.ShapeDtypeStruct((N, NHID), jnp.float32),
)


def _mid_body(acc_ref, b1_ref, w2_ref, o_ref):
    h = jax.nn.relu(acc_ref[0] + acc_ref[1] + b1_ref[...])
    o_ref[...] = jnp.dot(h, w2_ref[...], preferred_element_type=jnp.float32)


_mid = pl.pallas_call(
    _mid_body,
    grid=(GRID,),
    in_specs=[
        pl.BlockSpec((NC, ROW_BLK, NHID), lambda i: (0, i, 0)),
        pl.BlockSpec((1, NHID), lambda i: (0, 0)),
        pl.BlockSpec((NHID, NHID), lambda i: (0, 0)),
    ],
    out_specs=pl.BlockSpec((ROW_BLK, NHID), lambda i: (i, 0)),
    out_shape=jax.ShapeDtypeStruct((N, NHID), jnp.float32),
)


def _post_body(acc_ref, b2_ref, tf_ref, l1w_ref, l1b_ref, l2w_ref, l2b_ref,
               l3w_ref, l3b_ref, o_ref):
    out3 = acc_ref[0] + acc_ref[1] + b2_ref[...]
    cat = jnp.concatenate([out3, tf_ref[...]], axis=1)
    h = jax.nn.relu(jnp.dot(cat, l1w_ref[...],
                            preferred_element_type=jnp.float32) + l1b_ref[...])
    h = jax.nn.relu(jnp.dot(h, l2w_ref[...],
                            preferred_element_type=jnp.float32) + l2b_ref[...])
    o = jnp.dot(h, l3w_ref[...],
                preferred_element_type=jnp.float32) + l3b_ref[...]
    m = jnp.max(o, axis=1, keepdims=True)
    s = jnp.sum(jnp.exp(o - m), axis=1, keepdims=True)
    o_ref[...] = o - m - jnp.log(s)


_post = pl.pallas_call(
    _post_body,
    grid=(GRID,),
    in_specs=[
        pl.BlockSpec((NC, ROW_BLK, NHID), lambda i: (0, i, 0)),
        pl.BlockSpec((1, NHID), lambda i: (0, 0)),
        pl.BlockSpec((ROW_BLK, NFEAT), lambda i: (i, 0)),
        pl.BlockSpec((NCAT, NCAT), lambda i: (0, 0)),
        pl.BlockSpec((1, NCAT), lambda i: (0, 0)),
        pl.BlockSpec((NCAT, NCAT), lambda i: (0, 0)),
        pl.BlockSpec((1, NCAT), lambda i: (0, 0)),
        pl.BlockSpec((NCAT, NFEAT), lambda i: (0, 0)),
        pl.BlockSpec((1, NFEAT), lambda i: (0, 0)),
    ],
    out_specs=pl.BlockSpec((ROW_BLK, NFEAT), lambda i: (i, 0)),
    out_shape=jax.ShapeDtypeStruct((N, NFEAT), jnp.float32),
)


def kernel(x, target_feats, adj, W1, b1, W2, b2, L1W, L1b, L2W, L2b, L3W, L3b):
    zeros = jnp.zeros((SLAB, NFEAT), jnp.float32)

    support1 = _pre(x, W1)
    acc1 = _sc_agg0(support1, adj, zeros)
    support2 = _mid(acc1, b1.reshape(1, NHID), W2)
    acc2 = _sc_agg1(support2, adj, zeros)
    return _post(acc2, b2.reshape(1, NHID), target_feats,
                 L1W, L1b.reshape(1, NCAT), L2W, L2b.reshape(1, NCAT),
                 L3W, L3b.reshape(1, NFEAT))


# speedup vs baseline: 13.4018x; 1.0028x over previous
import functools

import jax
import jax.numpy as jnp
from jax import lax
from jax.experimental import pallas as pl
from jax.experimental.pallas import tpu as pltpu
from jax.experimental.pallas import tpu_sc as plsc

N = 10000
E = 320000
NFEAT = 128
NHID = 128
NCAT = NFEAT + NHID

CHUNK = 128
NCHUNK = E // CHUNK
NC = 2
NS = 16
NW = NC * NS
BASE_CHUNKS = NCHUNK // NW
EXTRA_CHUNKS = NCHUNK - BASE_CHUNKS * NW
SLAB = (N // NS) // 8 * 8
TAIL = N - NS * SLAB

ROW_BLK = 1000
GRID = N // ROW_BLK


NBUF = 3


def _sc_agg_body(lay, support_hbm, adj_hbm, zeros_hbm, out_hbm,
                 sidx0, sidx1, sidx2, didx0, didx1, didx2,
                 rows0, rows1, rows2, acc_sh,
                 sem_a0, sem_a1, sem_a2, sem_b0, sem_b1, sem_b2,
                 sem_g0, sem_g1, sem_g2, sem_s0, sem_s1, sem_s2):
    src_hbm = adj_hbm.at[lay, 0]
    dst_hbm = adj_hbm.at[lay, 1]
    cid = lax.axis_index("c")
    sid = lax.axis_index("s")
    wid = sid * NC + cid
    sidx = (sidx0, sidx1, sidx2)
    didx = (didx0, didx1, didx2)
    rows = (rows0, rows1, rows2)
    sem_a = (sem_a0, sem_a1, sem_a2)
    sem_b = (sem_b0, sem_b1, sem_b2)
    sem_g = (sem_g0, sem_g1, sem_g2)
    sem_s = (sem_s0, sem_s1, sem_s2)

    cstart = wid * BASE_CHUNKS + jnp.minimum(wid, EXTRA_CHUNKS)

    def _issue_idx(kk, b):
        pltpu.async_copy(src_hbm.at[pl.ds((cstart + kk) * CHUNK, CHUNK)],
                         sidx[b], sem_a[b])
        pltpu.async_copy(dst_hbm.at[pl.ds((cstart + kk) * CHUNK, CHUNK)],
                         didx[b], sem_b[b])

    def _issue_gather(b):
        pltpu.make_async_copy(src_hbm.at[pl.ds(0, CHUNK)], sidx[b],
                              sem_a[b]).wait()
        pltpu.async_copy(support_hbm.at[sidx[b]], rows[b], sem_g[b])

    _issue_idx(0, 0)
    _issue_idx(1, 1)
    _issue_gather(0)

    pltpu.sync_copy(zeros_hbm, acc_sh.at[pl.ds(sid * SLAB, SLAB)])

    @pl.when(sid == 0)
    def _zero_tail():
        pltpu.sync_copy(zeros_hbm.at[pl.ds(0, TAIL)],
                        acc_sh.at[pl.ds(NS * SLAB, TAIL)])

    plsc.subcore_barrier()

    @pl.loop(0, BASE_CHUNKS, step=NBUF)
    def _ring(g):
        for b in range(NBUF):
            k = g + b
            b1 = (b + 1) % NBUF
            b2 = (b + 2) % NBUF

            @pl.when(k + 1 < BASE_CHUNKS)
            def _ahead_gather():
                _issue_gather(b1)

            @pl.when(k >= 1)
            def _drain():
                pltpu.make_async_copy(
                    rows[b2], acc_sh.at[didx[b2]], sem_s[b2]).wait()

            @pl.when(k + 2 < BASE_CHUNKS)
            def _ahead_idx():
                _issue_idx(k + 2, b2)

            pltpu.make_async_copy(support_hbm.at[sidx[b]], rows[b],
                                  sem_g[b]).wait()
            pltpu.make_async_copy(dst_hbm.at[pl.ds(0, CHUNK)], didx[b],
                                  sem_b[b]).wait()
            pltpu.async_copy(rows[b], acc_sh.at[didx[b]], sem_s[b], add=True)

    bl = (BASE_CHUNKS - 1) % NBUF
    pltpu.make_async_copy(rows[bl], acc_sh.at[didx[bl]], sem_s[bl]).wait()

    @pl.when(wid < EXTRA_CHUNKS)
    def _extra():
        chunk = cstart + BASE_CHUNKS
        pltpu.sync_copy(src_hbm.at[pl.ds(chunk * CHUNK, CHUNK)], sidx0)
        pltpu.sync_copy(dst_hbm.at[pl.ds(chunk * CHUNK, CHUNK)], didx0)
        pltpu.async_copy(support_hbm.at[sidx0], rows0, sem_g0).wait()
        pltpu.sync_copy(rows0, acc_sh.at[didx0], add=True)

    plsc.subcore_barrier()

    pltpu.sync_copy(acc_sh.at[pl.ds(sid * SLAB, SLAB)],
                    out_hbm.at[cid, pl.ds(sid * SLAB, SLAB)])

    @pl.when(sid == 0)
    def _write_tail():
        pltpu.sync_copy(acc_sh.at[pl.ds(NS * SLAB, TAIL)],
                        out_hbm.at[cid, pl.ds(NS * SLAB, TAIL)])


def _make_sc_agg(lay):
    return pl.kernel(
        functools.partial(_sc_agg_body, lay),
        out_type=jax.ShapeDtypeStruct((NC, N, NFEAT), jnp.float32),
        mesh=plsc.VectorSubcoreMesh(core_axis_name="c", subcore_axis_name="s"),
        scratch_types=(
            [pltpu.VMEM((CHUNK,), jnp.int32)] * 6
            + [pltpu.VMEM((CHUNK, NFEAT), jnp.float32)] * 3
            + [pltpu.VMEM_SHARED((N, NFEAT), jnp.float32)]
            + [pltpu.SemaphoreType.DMA] * 12
        ),
    )


_sc_agg0 = _make_sc_agg(0)
_sc_agg1 = _make_sc_agg(1)


def _pre_body(x_ref, w_ref, o_ref):
    o_ref[...] = jnp.dot(x_ref[...], w_ref[...],
                         preferred_element_type=jnp.float32,
                         precision=lax.Precision.DEFAULT)


_pre = pl.pallas_call(
    _pre_body,
    grid=(GRID,),
    in_specs=[
        pl.BlockSpec((ROW_BLK, NFEAT), lambda i: (i, 0)),
        pl.BlockSpec((NFEAT, NHID), lambda i: (0, 0)),
    ],
    out_specs=pl.BlockSpec((ROW_BLK, NHID), lambda i: (i, 0)),
    out_shape=jax.ShapeDtypeStruct((N, NHID), jnp.float32),
)


def _mid_body(acc_ref, b1_ref, w2_ref, o_ref):
    h = jax.nn.relu(acc_ref[0] + acc_ref[1] + b1_ref[...])
    o_ref[...] = jnp.dot(h, w2_ref[...], preferred_element_type=jnp.float32,
                         precision=lax.Precision.DEFAULT)


_mid = pl.pallas_call(
    _mid_body,
    grid=(GRID,),
    in_specs=[
        pl.BlockSpec((NC, ROW_BLK, NHID), lambda i: (0, i, 0)),
        pl.BlockSpec((1, NHID), lambda i: (0, 0)),
        pl.BlockSpec((NHID, NHID), lambda i: (0, 0)),
    ],
    out_specs=pl.BlockSpec((ROW_BLK, NHID), lambda i: (i, 0)),
    out_shape=jax.ShapeDtypeStruct((N, NHID), jnp.float32),
)


def _post_body(acc_ref, b2_ref, tf_ref, l1w_ref, l1b_ref, l2w_ref, l2b_ref,
               l3w_ref, l3b_ref, o_ref):
    out3 = acc_ref[0] + acc_ref[1] + b2_ref[...]
    cat = jnp.concatenate([out3, tf_ref[...]], axis=1)
    h = jax.nn.relu(jnp.dot(cat, l1w_ref[...],
                            preferred_element_type=jnp.float32,
                         precision=lax.Precision.DEFAULT) + l1b_ref[...])
    h = jax.nn.relu(jnp.dot(h, l2w_ref[...],
                            preferred_element_type=jnp.float32,
                         precision=lax.Precision.DEFAULT) + l2b_ref[...])
    o = jnp.dot(h, l3w_ref[...],
                preferred_element_type=jnp.float32,
                         precision=lax.Precision.DEFAULT) + l3b_ref[...]
    m = jnp.max(o, axis=1, keepdims=True)
    s = jnp.sum(jnp.exp(o - m), axis=1, keepdims=True)
    o_ref[...] = o - m - jnp.log(s)


_post = pl.pallas_call(
    _post_body,
    grid=(GRID,),
    in_specs=[
        pl.BlockSpec((NC, ROW_BLK, NHID), lambda i: (0, i, 0)),
        pl.BlockSpec((1, NHID), lambda i: (0, 0)),
        pl.BlockSpec((ROW_BLK, NFEAT), lambda i: (i, 0)),
        pl.BlockSpec((NCAT, NCAT), lambda i: (0, 0)),
        pl.BlockSpec((1, NCAT), lambda i: (0, 0)),
        pl.BlockSpec((NCAT, NCAT), lambda i: (0, 0)),
        pl.BlockSpec((1, NCAT), lambda i: (0, 0)),
        pl.BlockSpec((NCAT, NFEAT), lambda i: (0, 0)),
        pl.BlockSpec((1, NFEAT), lambda i: (0, 0)),
    ],
    out_specs=pl.BlockSpec((ROW_BLK, NFEAT), lambda i: (i, 0)),
    out_shape=jax.ShapeDtypeStruct((N, NFEAT), jnp.float32),
)


def kernel(x, target_feats, adj, W1, b1, W2, b2, L1W, L1b, L2W, L2b, L3W, L3b):
    zeros = jnp.zeros((SLAB, NFEAT), jnp.float32)

    support1 = _pre(x, W1)
    acc1 = _sc_agg0(support1, adj, zeros)
    support2 = _mid(acc1, b1.reshape(1, NHID), W2)
    acc2 = _sc_agg1(support2, adj, zeros)
    return _post(acc2, b2.reshape(1, NHID), target_feats,
                 L1W, L1b.reshape(1, NCAT), L2W, L2b.reshape(1, NCAT),
                 L3W, L3b.reshape(1, NFEAT))


# speedup vs baseline: 13.8774x; 1.0355x over previous
import functools

import jax
import jax.numpy as jnp
from jax import lax
from jax.experimental import pallas as pl
from jax.experimental.pallas import tpu as pltpu
from jax.experimental.pallas import tpu_sc as plsc

N = 10000
E = 320000
NFEAT = 128
NHID = 128
NCAT = NFEAT + NHID

CHUNK = 128
NCHUNK = E // CHUNK
NC = 2
NS = 16
NW = NC * NS
BASE_CHUNKS = NCHUNK // NW
EXTRA_CHUNKS = NCHUNK - BASE_CHUNKS * NW
SLAB = (N // NS) // 8 * 8
TAIL = N - NS * SLAB

ROW_BLK = 2000
GRID = N // ROW_BLK


NBUF = 3


def _sc_agg_body(lay, support_hbm, adj_hbm, zeros_hbm, out_hbm,
                 sidx0, sidx1, sidx2, didx0, didx1, didx2,
                 rows0, rows1, rows2, acc_sh,
                 sem_a0, sem_a1, sem_a2, sem_b0, sem_b1, sem_b2,
                 sem_g0, sem_g1, sem_g2, sem_s0, sem_s1, sem_s2):
    src_hbm = adj_hbm.at[lay, 0]
    dst_hbm = adj_hbm.at[lay, 1]
    cid = lax.axis_index("c")
    sid = lax.axis_index("s")
    wid = sid * NC + cid
    sidx = (sidx0, sidx1, sidx2)
    didx = (didx0, didx1, didx2)
    rows = (rows0, rows1, rows2)
    sem_a = (sem_a0, sem_a1, sem_a2)
    sem_b = (sem_b0, sem_b1, sem_b2)
    sem_g = (sem_g0, sem_g1, sem_g2)
    sem_s = (sem_s0, sem_s1, sem_s2)

    cstart = wid * BASE_CHUNKS + jnp.minimum(wid, EXTRA_CHUNKS)

    def _issue_idx(kk, b):
        pltpu.async_copy(src_hbm.at[pl.ds((cstart + kk) * CHUNK, CHUNK)],
                         sidx[b], sem_a[b])
        pltpu.async_copy(dst_hbm.at[pl.ds((cstart + kk) * CHUNK, CHUNK)],
                         didx[b], sem_b[b])

    def _issue_gather(b):
        pltpu.make_async_copy(src_hbm.at[pl.ds(0, CHUNK)], sidx[b],
                              sem_a[b]).wait()
        pltpu.async_copy(support_hbm.at[sidx[b]], rows[b], sem_g[b])

    _issue_idx(0, 0)
    _issue_idx(1, 1)
    _issue_gather(0)

    pltpu.sync_copy(zeros_hbm, acc_sh.at[pl.ds(sid * SLAB, SLAB)])

    @pl.when(sid == 0)
    def _zero_tail():
        pltpu.sync_copy(zeros_hbm.at[pl.ds(0, TAIL)],
                        acc_sh.at[pl.ds(NS * SLAB, TAIL)])

    plsc.subcore_barrier()

    @pl.loop(0, BASE_CHUNKS, step=NBUF)
    def _ring(g):
        for b in range(NBUF):
            k = g + b
            b1 = (b + 1) % NBUF
            b2 = (b + 2) % NBUF

            @pl.when(k + 1 < BASE_CHUNKS)
            def _ahead_gather():
                _issue_gather(b1)

            @pl.when(k >= 1)
            def _drain():
                pltpu.make_async_copy(
                    rows[b2], acc_sh.at[didx[b2]], sem_s[b2]).wait()

            @pl.when(k + 2 < BASE_CHUNKS)
            def _ahead_idx():
                _issue_idx(k + 2, b2)

            pltpu.make_async_copy(support_hbm.at[sidx[b]], rows[b],
                                  sem_g[b]).wait()
            pltpu.make_async_copy(dst_hbm.at[pl.ds(0, CHUNK)], didx[b],
                                  sem_b[b]).wait()
            pltpu.async_copy(rows[b], acc_sh.at[didx[b]], sem_s[b], add=True)

    bl = (BASE_CHUNKS - 1) % NBUF
    pltpu.make_async_copy(rows[bl], acc_sh.at[didx[bl]], sem_s[bl]).wait()

    @pl.when(wid < EXTRA_CHUNKS)
    def _extra():
        chunk = cstart + BASE_CHUNKS
        pltpu.sync_copy(src_hbm.at[pl.ds(chunk * CHUNK, CHUNK)], sidx0)
        pltpu.sync_copy(dst_hbm.at[pl.ds(chunk * CHUNK, CHUNK)], didx0)
        pltpu.async_copy(support_hbm.at[sidx0], rows0, sem_g0).wait()
        pltpu.sync_copy(rows0, acc_sh.at[didx0], add=True)

    plsc.subcore_barrier()

    pltpu.sync_copy(acc_sh.at[pl.ds(sid * SLAB, SLAB)],
                    out_hbm.at[cid, pl.ds(sid * SLAB, SLAB)])

    @pl.when(sid == 0)
    def _write_tail():
        pltpu.sync_copy(acc_sh.at[pl.ds(NS * SLAB, TAIL)],
                        out_hbm.at[cid, pl.ds(NS * SLAB, TAIL)])


def _make_sc_agg(lay):
    return pl.kernel(
        functools.partial(_sc_agg_body, lay),
        out_type=jax.ShapeDtypeStruct((NC, N, NFEAT), jnp.float32),
        mesh=plsc.VectorSubcoreMesh(core_axis_name="c", subcore_axis_name="s"),
        scratch_types=(
            [pltpu.VMEM((CHUNK,), jnp.int32)] * 6
            + [pltpu.VMEM((CHUNK, NFEAT), jnp.float32)] * 3
            + [pltpu.VMEM_SHARED((N, NFEAT), jnp.float32)]
            + [pltpu.SemaphoreType.DMA] * 12
        ),
    )


_sc_agg0 = _make_sc_agg(0)
_sc_agg1 = _make_sc_agg(1)


def _pre_body(x_ref, w_ref, o_ref):
    o_ref[...] = jnp.dot(x_ref[...], w_ref[...],
                         preferred_element_type=jnp.float32,
                         precision=lax.Precision.DEFAULT)


_pre = pl.pallas_call(
    _pre_body,
    grid=(GRID,),
    in_specs=[
        pl.BlockSpec((ROW_BLK, NFEAT), lambda i: (i, 0)),
        pl.BlockSpec((NFEAT, NHID), lambda i: (0, 0)),
    ],
    out_specs=pl.BlockSpec((ROW_BLK, NHID), lambda i: (i, 0)),
    out_shape=jax.ShapeDtypeStruct((N, NHID), jnp.float32),
)


def _mid_body(acc_ref, b1_ref, w2_ref, o_ref):
    h = jax.nn.relu(acc_ref[0] + acc_ref[1] + b1_ref[...])
    o_ref[...] = jnp.dot(h, w2_ref[...], preferred_element_type=jnp.float32,
                         precision=lax.Precision.DEFAULT)


_mid = pl.pallas_call(
    _mid_body,
    grid=(GRID,),
    in_specs=[
        pl.BlockSpec((NC, ROW_BLK, NHID), lambda i: (0, i, 0)),
        pl.BlockSpec((1, NHID), lambda i: (0, 0)),
        pl.BlockSpec((NHID, NHID), lambda i: (0, 0)),
    ],
    out_specs=pl.BlockSpec((ROW_BLK, NHID), lambda i: (i, 0)),
    out_shape=jax.ShapeDtypeStruct((N, NHID), jnp.float32),
)


def _post_body(acc_ref, b2_ref, tf_ref, l1w_ref, l1b_ref, l2w_ref, l2b_ref,
               l3w_ref, l3b_ref, o_ref):
    out3 = acc_ref[0] + acc_ref[1] + b2_ref[...]
    cat = jnp.concatenate([out3, tf_ref[...]], axis=1)
    h = jax.nn.relu(jnp.dot(cat, l1w_ref[...],
                            preferred_element_type=jnp.float32,
                         precision=lax.Precision.DEFAULT) + l1b_ref[...])
    h = jax.nn.relu(jnp.dot(h, l2w_ref[...],
                            preferred_element_type=jnp.float32,
                         precision=lax.Precision.DEFAULT) + l2b_ref[...])
    o = jnp.dot(h, l3w_ref[...],
                preferred_element_type=jnp.float32,
                         precision=lax.Precision.DEFAULT) + l3b_ref[...]
    m = jnp.max(o, axis=1, keepdims=True)
    s = jnp.sum(jnp.exp(o - m), axis=1, keepdims=True)
    o_ref[...] = o - m - jnp.log(s)


_post = pl.pallas_call(
    _post_body,
    grid=(GRID,),
    in_specs=[
        pl.BlockSpec((NC, ROW_BLK, NHID), lambda i: (0, i, 0)),
        pl.BlockSpec((1, NHID), lambda i: (0, 0)),
        pl.BlockSpec((ROW_BLK, NFEAT), lambda i: (i, 0)),
        pl.BlockSpec((NCAT, NCAT), lambda i: (0, 0)),
        pl.BlockSpec((1, NCAT), lambda i: (0, 0)),
        pl.BlockSpec((NCAT, NCAT), lambda i: (0, 0)),
        pl.BlockSpec((1, NCAT), lambda i: (0, 0)),
        pl.BlockSpec((NCAT, NFEAT), lambda i: (0, 0)),
        pl.BlockSpec((1, NFEAT), lambda i: (0, 0)),
    ],
    out_specs=pl.BlockSpec((ROW_BLK, NFEAT), lambda i: (i, 0)),
    out_shape=jax.ShapeDtypeStruct((N, NFEAT), jnp.float32),
)


def kernel(x, target_feats, adj, W1, b1, W2, b2, L1W, L1b, L2W, L2b, L3W, L3b):
    zeros = jnp.zeros((SLAB, NFEAT), jnp.float32)

    support1 = _pre(x, W1)
    acc1 = _sc_agg0(support1, adj, zeros)
    support2 = _mid(acc1, b1.reshape(1, NHID), W2)
    acc2 = _sc_agg1(support2, adj, zeros)
    return _post(acc2, b2.reshape(1, NHID), target_feats,
                 L1W, L1b.reshape(1, NCAT), L2W, L2b.reshape(1, NCAT),
                 L3W, L3b.reshape(1, NFEAT))


# speedup vs baseline: 14.2102x; 1.0240x over previous
import functools

import jax
import jax.numpy as jnp
from jax import lax
from jax.experimental import pallas as pl
from jax.experimental.pallas import tpu as pltpu
from jax.experimental.pallas import tpu_sc as plsc

N = 10000
E = 320000
NFEAT = 128
NHID = 128
NCAT = NFEAT + NHID

CHUNK = 128
NCHUNK = E // CHUNK
NC = 2
NS = 16
NW = NC * NS
BASE_CHUNKS = NCHUNK // NW
EXTRA_CHUNKS = NCHUNK - BASE_CHUNKS * NW
SLAB = (N // NS) // 8 * 8
TAIL = N - NS * SLAB

ROW_BLK = 5000
GRID = N // ROW_BLK


NBUF = 3


def _sc_agg_body(lay, support_hbm, adj_hbm, zeros_hbm, out_hbm,
                 sidx0, sidx1, sidx2, didx0, didx1, didx2,
                 rows0, rows1, rows2, acc_sh,
                 sem_a0, sem_a1, sem_a2, sem_b0, sem_b1, sem_b2,
                 sem_g0, sem_g1, sem_g2, sem_s0, sem_s1, sem_s2):
    src_hbm = adj_hbm.at[lay, 0]
    dst_hbm = adj_hbm.at[lay, 1]
    cid = lax.axis_index("c")
    sid = lax.axis_index("s")
    wid = sid * NC + cid
    sidx = (sidx0, sidx1, sidx2)
    didx = (didx0, didx1, didx2)
    rows = (rows0, rows1, rows2)
    sem_a = (sem_a0, sem_a1, sem_a2)
    sem_b = (sem_b0, sem_b1, sem_b2)
    sem_g = (sem_g0, sem_g1, sem_g2)
    sem_s = (sem_s0, sem_s1, sem_s2)

    cstart = wid * BASE_CHUNKS + jnp.minimum(wid, EXTRA_CHUNKS)

    def _issue_idx(kk, b):
        pltpu.async_copy(src_hbm.at[pl.ds((cstart + kk) * CHUNK, CHUNK)],
                         sidx[b], sem_a[b])
        pltpu.async_copy(dst_hbm.at[pl.ds((cstart + kk) * CHUNK, CHUNK)],
                         didx[b], sem_b[b])

    def _issue_gather(b):
        pltpu.make_async_copy(src_hbm.at[pl.ds(0, CHUNK)], sidx[b],
                              sem_a[b]).wait()
        pltpu.async_copy(support_hbm.at[sidx[b]], rows[b], sem_g[b])

    _issue_idx(0, 0)
    _issue_idx(1, 1)
    _issue_gather(0)

    pltpu.sync_copy(zeros_hbm, acc_sh.at[pl.ds(sid * SLAB, SLAB)])

    @pl.when(sid == 0)
    def _zero_tail():
        pltpu.sync_copy(zeros_hbm.at[pl.ds(0, TAIL)],
                        acc_sh.at[pl.ds(NS * SLAB, TAIL)])

    plsc.subcore_barrier()

    @pl.loop(0, BASE_CHUNKS, step=NBUF)
    def _ring(g):
        for b in range(NBUF):
            k = g + b
            b1 = (b + 1) % NBUF
            b2 = (b + 2) % NBUF

            @pl.when(k + 1 < BASE_CHUNKS)
            def _ahead_gather():
                _issue_gather(b1)

            @pl.when(k >= 1)
            def _drain():
                pltpu.make_async_copy(
                    rows[b2], acc_sh.at[didx[b2]], sem_s[b2]).wait()

            @pl.when(k + 2 < BASE_CHUNKS)
            def _ahead_idx():
                _issue_idx(k + 2, b2)

            pltpu.make_async_copy(support_hbm.at[sidx[b]], rows[b],
                                  sem_g[b]).wait()
            pltpu.make_async_copy(dst_hbm.at[pl.ds(0, CHUNK)], didx[b],
                                  sem_b[b]).wait()
            pltpu.async_copy(rows[b], acc_sh.at[didx[b]], sem_s[b], add=True)

    bl = (BASE_CHUNKS - 1) % NBUF
    pltpu.make_async_copy(rows[bl], acc_sh.at[didx[bl]], sem_s[bl]).wait()

    @pl.when(wid < EXTRA_CHUNKS)
    def _extra():
        chunk = cstart + BASE_CHUNKS
        pltpu.sync_copy(src_hbm.at[pl.ds(chunk * CHUNK, CHUNK)], sidx0)
        pltpu.sync_copy(dst_hbm.at[pl.ds(chunk * CHUNK, CHUNK)], didx0)
        pltpu.async_copy(support_hbm.at[sidx0], rows0, sem_g0).wait()
        pltpu.sync_copy(rows0, acc_sh.at[didx0], add=True)

    plsc.subcore_barrier()

    pltpu.sync_copy(acc_sh.at[pl.ds(sid * SLAB, SLAB)],
                    out_hbm.at[cid, pl.ds(sid * SLAB, SLAB)])

    @pl.when(sid == 0)
    def _write_tail():
        pltpu.sync_copy(acc_sh.at[pl.ds(NS * SLAB, TAIL)],
                        out_hbm.at[cid, pl.ds(NS * SLAB, TAIL)])


def _make_sc_agg(lay):
    return pl.kernel(
        functools.partial(_sc_agg_body, lay),
        out_type=jax.ShapeDtypeStruct((NC, N, NFEAT), jnp.float32),
        mesh=plsc.VectorSubcoreMesh(core_axis_name="c", subcore_axis_name="s"),
        scratch_types=(
            [pltpu.VMEM((CHUNK,), jnp.int32)] * 6
            + [pltpu.VMEM((CHUNK, NFEAT), jnp.float32)] * 3
            + [pltpu.VMEM_SHARED((N, NFEAT), jnp.float32)]
            + [pltpu.SemaphoreType.DMA] * 12
        ),
    )


_sc_agg0 = _make_sc_agg(0)
_sc_agg1 = _make_sc_agg(1)


def _pre_body(x_ref, w_ref, o_ref):
    o_ref[...] = jnp.dot(x_ref[...], w_ref[...],
                         preferred_element_type=jnp.float32,
                         precision=lax.Precision.DEFAULT)


_pre = pl.pallas_call(
    _pre_body,
    grid=(GRID,),
    in_specs=[
        pl.BlockSpec((ROW_BLK, NFEAT), lambda i: (i, 0)),
        pl.BlockSpec((NFEAT, NHID), lambda i: (0, 0)),
    ],
    out_specs=pl.BlockSpec((ROW_BLK, NHID), lambda i: (i, 0)),
    out_shape=jax.ShapeDtypeStruct((N, NHID), jnp.float32),
)


def _mid_body(acc_ref, b1_ref, w2_ref, o_ref):
    h = jax.nn.relu(acc_ref[0] + acc_ref[1] + b1_ref[...])
    o_ref[...] = jnp.dot(h, w2_ref[...], preferred_element_type=jnp.float32,
                         precision=lax.Precision.DEFAULT)


_mid = pl.pallas_call(
    _mid_body,
    grid=(GRID,),
    in_specs=[
        pl.BlockSpec((NC, ROW_BLK, NHID), lambda i: (0, i, 0)),
        pl.BlockSpec((1, NHID), lambda i: (0, 0)),
        pl.BlockSpec((NHID, NHID), lambda i: (0, 0)),
    ],
    out_specs=pl.BlockSpec((ROW_BLK, NHID), lambda i: (i, 0)),
    out_shape=jax.ShapeDtypeStruct((N, NHID), jnp.float32),
)


def _post_body(acc_ref, b2_ref, tf_ref, l1w_ref, l1b_ref, l2w_ref, l2b_ref,
               l3w_ref, l3b_ref, o_ref):
    out3 = acc_ref[0] + acc_ref[1] + b2_ref[...]
    cat = jnp.concatenate([out3, tf_ref[...]], axis=1)
    h = jax.nn.relu(jnp.dot(cat, l1w_ref[...],
                            preferred_element_type=jnp.float32,
                         precision=lax.Precision.DEFAULT) + l1b_ref[...])
    h = jax.nn.relu(jnp.dot(h, l2w_ref[...],
                            preferred_element_type=jnp.float32,
                         precision=lax.Precision.DEFAULT) + l2b_ref[...])
    o = jnp.dot(h, l3w_ref[...],
                preferred_element_type=jnp.float32,
                         precision=lax.Precision.DEFAULT) + l3b_ref[...]
    m = jnp.max(o, axis=1, keepdims=True)
    s = jnp.sum(jnp.exp(o - m), axis=1, keepdims=True)
    o_ref[...] = o - m - jnp.log(s)


_post = pl.pallas_call(
    _post_body,
    grid=(GRID,),
    in_specs=[
        pl.BlockSpec((NC, ROW_BLK, NHID), lambda i: (0, i, 0)),
        pl.BlockSpec((1, NHID), lambda i: (0, 0)),
        pl.BlockSpec((ROW_BLK, NFEAT), lambda i: (i, 0)),
        pl.BlockSpec((NCAT, NCAT), lambda i: (0, 0)),
        pl.BlockSpec((1, NCAT), lambda i: (0, 0)),
        pl.BlockSpec((NCAT, NCAT), lambda i: (0, 0)),
        pl.BlockSpec((1, NCAT), lambda i: (0, 0)),
        pl.BlockSpec((NCAT, NFEAT), lambda i: (0, 0)),
        pl.BlockSpec((1, NFEAT), lambda i: (0, 0)),
    ],
    out_specs=pl.BlockSpec((ROW_BLK, NFEAT), lambda i: (i, 0)),
    out_shape=jax.ShapeDtypeStruct((N, NFEAT), jnp.float32),
)


def kernel(x, target_feats, adj, W1, b1, W2, b2, L1W, L1b, L2W, L2b, L3W, L3b):
    zeros = jnp.zeros((SLAB, NFEAT), jnp.float32)

    support1 = _pre(x, W1)
    acc1 = _sc_agg0(support1, adj, zeros)
    support2 = _mid(acc1, b1.reshape(1, NHID), W2)
    acc2 = _sc_agg1(support2, adj, zeros)
    return _post(acc2, b2.reshape(1, NHID), target_feats,
                 L1W, L1b.reshape(1, NCAT), L2W, L2b.reshape(1, NCAT),
                 L3W, L3b.reshape(1, NFEAT))
